# baseline (speedup 1.0000x reference)
import functools

import jax
import jax.numpy as jnp
from jax.experimental import pallas as pl
from jax.experimental.pallas import tpu as pltpu

_N = 349184
_NC = 16
_K = max(int(_N * 0.01), 2)
_NBLK = 124
_BR = _N // _NBLK


def _dense_body(t_cls_ref, s_cls_ref, t_cent_ref, scores_ref, joint_ref,
                delta_ref, acc_ref):
    t = t_cls_ref[...]
    s = s_cls_ref[...]
    tc = t_cent_ref[...]
    mx = jnp.max(t, axis=1, keepdims=True)
    sc = jax.nn.sigmoid(mx)
    scores_ref[...] = sc
    joint_ref[...] = jax.nn.sigmoid(tc) * sc
    s_sig = jax.nn.sigmoid(s)
    t_sig = jax.nn.sigmoid(t)
    p = jnp.clip(s_sig, 1e-12, 1.0 - 1e-12)
    logp = jnp.log(p)
    log1mp = jnp.log(1.0 - p)
    ln = -log1mp * (s_sig * s_sig)
    d = t_sig - s_sig
    lp = -(t_sig * logp + (1.0 - t_sig) * log1mp) * (d * d)

    @pl.when(pl.program_id(0) == 0)
    def _():
        acc_ref[0, 0] = 0.0
        acc_ref[0, 1] = 0.0

    delta_ref[...] = jnp.sum(lp - ln, axis=1, keepdims=True)
    acc_ref[0, 0] += jnp.sum(ln)
    acc_ref[0, 1] += jnp.sum(sc)


@functools.partial(jax.jit, static_argnums=())
def _dense_pass(t_cls, s_cls, t_cent):
    grid = (_NBLK,)
    out = pl.pallas_call(
        _dense_body,
        grid=grid,
        in_specs=[
            pl.BlockSpec((_BR, _NC), lambda i: (i, 0)),
            pl.BlockSpec((_BR, _NC), lambda i: (i, 0)),
            pl.BlockSpec((_BR, 1), lambda i: (i, 0)),
        ],
        out_specs=[
            pl.BlockSpec((_BR, 1), lambda i: (i, 0)),
            pl.BlockSpec((_BR, 1), lambda i: (i, 0)),
            pl.BlockSpec((_BR, 1), lambda i: (i, 0)),
            pl.BlockSpec(memory_space=pltpu.SMEM),
        ],
        out_shape=[
            jax.ShapeDtypeStruct((_N, 1), jnp.float32),
            jax.ShapeDtypeStruct((_N, 1), jnp.float32),
            jax.ShapeDtypeStruct((_N, 1), jnp.float32),
            jax.ShapeDtypeStruct((1, 2), jnp.float32),
        ],
    )(t_cls, s_cls, t_cent.reshape(_N, 1))
    return out


def _box2corners(box):
    x, y, w, h, a = (box[..., i] for i in range(5))
    dx = jnp.array([0.5, -0.5, -0.5, 0.5], dtype=box.dtype) * w[..., None]
    dy = jnp.array([0.5, 0.5, -0.5, -0.5], dtype=box.dtype) * h[..., None]
    c = jnp.cos(a)[..., None]
    s = jnp.sin(a)[..., None]
    return jnp.stack([c * dx - s * dy + x[..., None],
                      s * dx + c * dy + y[..., None]], axis=-1)


def _edge_intersections(c1, c2):
    P = c1.shape[0]
    p1 = c1[:, :, None, :]
    r = (jnp.roll(c1, -1, axis=1) - c1)[:, :, None, :]
    q1 = c2[:, None, :, :]
    s = (jnp.roll(c2, -1, axis=1) - c2)[:, None, :, :]
    den = r[..., 0] * s[..., 1] - r[..., 1] * s[..., 0]
    qp = q1 - p1
    t_num = qp[..., 0] * s[..., 1] - qp[..., 1] * s[..., 0]
    u_num = qp[..., 0] * r[..., 1] - qp[..., 1] * r[..., 0]
    safe = jnp.where(jnp.abs(den) > 1e-12, den, 1.0)
    t = t_num / safe
    u = u_num / safe
    valid = (jnp.abs(den) > 1e-12) & (t > 0) & (t < 1) & (u > 0) & (u < 1)
    pts = p1 + t[..., None] * r
    pts = jnp.where(valid[..., None], pts, 0.0)
    return pts.reshape(P, 16, 2), valid.reshape(P, 16)


def _points_in_box(pts, corners):
    a = corners[:, 0:1, :]
    ab = corners[:, 1:2, :] - a
    ad = corners[:, 3:4, :] - a
    ap = pts - a
    pab = (ap * ab).sum(-1)
    pad = (ap * ad).sum(-1)
    ab2 = (ab * ab).sum(-1)
    ad2 = (ad * ad).sum(-1)
    e = 1e-6
    return (pab > -e) & (pab < ab2 + e) & (pad > -e) & (pad < ad2 + e)


def _rotated_iou(b1, b2):
    c1 = _box2corners(b1)
    c2 = _box2corners(b2)
    ipts, ival = _edge_intersections(c1, c2)
    m1 = _points_in_box(c1, c2)
    m2 = _points_in_box(c2, c1)
    verts = jnp.concatenate([ipts, c1, c2], axis=1)
    mask = jnp.concatenate([ival, m1, m2], axis=1)
    nv = jnp.maximum(mask.sum(-1), 1)
    center = (verts * mask[..., None]).sum(1) / nv[..., None].astype(verts.dtype)
    rel = verts - center[:, None, :]
    ang = jnp.where(mask, jnp.arctan2(rel[..., 1], rel[..., 0]), 1e8)
    order = jnp.argsort(ang, axis=1)
    rel_s = jnp.take_along_axis(rel, order[..., None], axis=1)
    mask_s = jnp.take_along_axis(mask, order, axis=1)
    rel_p = jnp.where(mask_s[..., None], rel_s, rel_s[:, 0:1, :])
    nxt = jnp.roll(rel_p, -1, axis=1)
    cross = rel_p[..., 0] * nxt[..., 1] - rel_p[..., 1] * nxt[..., 0]
    inter = 0.5 * jnp.abs(cross.sum(-1))
    a1 = jnp.abs(b1[..., 2] * b1[..., 3])
    a2 = jnp.abs(b2[..., 2] * b2[..., 3])
    union = jnp.maximum(a1 + a2 - inter, 1e-12)
    return inter / union


def _bce(p, t):
    p = jnp.clip(p, 1e-12, 1.0 - 1e-12)
    return -(t * jnp.log(p) + (1.0 - t) * jnp.log(1.0 - p))


def kernel(t_cls_scores, t_bbox_preds, t_centernesses, s_cls_scores,
           s_bbox_preds, s_centernesses):
    scores, joint, delta, acc = _dense_pass(t_cls_scores, s_cls_scores,
                                            t_centernesses)
    t_scores = scores[:, 0]
    t_joint_scores = joint[:, 0]
    neg_sum = acc[0, 0]
    S_dps = acc[0, 1] / _N

    pos_vals, pos_inds = jax.lax.top_k(t_scores, _K)
    fg_num = pos_vals.sum()
    loss_cls_sum = neg_sum + delta[:, 0][pos_inds].sum()

    s_bbox_pos = s_bbox_preds[pos_inds]
    t_bbox_pos = t_bbox_preds[pos_inds]
    ious = jnp.maximum(_rotated_iou(s_bbox_pos, t_bbox_pos), 1e-6)
    loss_bbox = -jnp.log(ious)
    t_cent_pos = jax.nn.sigmoid(t_centernesses[pos_inds])
    s_cent_pos = jax.nn.sigmoid(s_centernesses[pos_inds])
    loss_centerness = _bce(s_cent_pos, t_cent_pos)
    unsup_loss_cls = loss_cls_sum / fg_num
    unsup_loss_bbox = (loss_bbox * t_cent_pos).mean()
    unsup_loss_centerness = loss_centerness.mean()
    return (unsup_loss_cls, unsup_loss_bbox, unsup_loss_centerness, S_dps,
            t_joint_scores)
